# baseline (speedup 1.0000x reference)
import functools

import jax
import jax.numpy as jnp
from jax import lax
from jax.experimental import pallas as pl
from jax.experimental.pallas import tpu as pltpu
from jax.experimental.pallas import tpu_sc as plsc

HIDDEN = 64
CHUNK = 800


def _embed(idx, table):
    n = idx.shape[0]
    info = plsc.get_sparse_core_info()
    nw = info.num_cores * info.num_subcores
    n_per_w = n // nw
    n_chunks = n_per_w // CHUNK
    mesh = plsc.VectorSubcoreMesh(core_axis_name="c", subcore_axis_name="s")

    @functools.partial(
        pl.kernel,
        mesh=mesh,
        out_type=jax.ShapeDtypeStruct((n, HIDDEN), jnp.float32),
        scratch_types=[
            pltpu.VMEM((2, CHUNK), jnp.int32),
            pltpu.VMEM((2, CHUNK, HIDDEN), jnp.float32),
            pltpu.SemaphoreType.DMA,
            pltpu.SemaphoreType.DMA,
            pltpu.SemaphoreType.DMA,
            pltpu.SemaphoreType.DMA,
        ],
        compiler_params=pltpu.CompilerParams(use_tc_tiling_on_sc=False),
    )
    def emb(idx_hbm, table_hbm, out_hbm, idx_v, rows_v, g0, g1, s0, s1):
        wid = lax.axis_index("s") * info.num_cores + lax.axis_index("c")
        base = wid * n_per_w
        gsem = (g0, g1)
        ssem = (s0, s1)

        def fire_gather(i, sl):
            off = base + i * CHUNK
            pltpu.sync_copy(idx_hbm.at[pl.ds(off, CHUNK)], idx_v.at[sl])
            return pltpu.async_copy(table_hbm.at[idx_v.at[sl]], rows_v.at[sl], gsem[sl])

        def fire_store(i, sl):
            off = base + i * CHUNK
            return pltpu.async_copy(rows_v.at[sl], out_hbm.at[pl.ds(off, CHUNK)], ssem[sl])

        gathers = [None] * n_chunks
        stores = [None] * n_chunks
        gathers[0] = fire_gather(0, 0)
        for i in range(n_chunks):
            sl = i % 2
            nsl = (i + 1) % 2
            if i + 1 < n_chunks:
                if i >= 1:
                    stores[i - 1].wait()
                gathers[i + 1] = fire_gather(i + 1, nsl)
            gathers[i].wait()
            stores[i] = fire_store(i, sl)
        stores[n_chunks - 1].wait()
        if n_chunks >= 2:
            stores[n_chunks - 2].wait()

    return emb(idx, table)


def kernel(tokens, embedding_table):
    b, s = tokens.shape
    idx = tokens.reshape(b * s).astype(jnp.int32)
    out = _embed(idx, embedding_table)
    return (tokens, out.reshape(b, s, HIDDEN))
